# baseline (speedup 1.0000x reference)
import functools

import jax
import jax.numpy as jnp
from jax import lax
from jax.experimental import pallas as pl
from jax.experimental.pallas import tpu as pltpu
from jax.experimental.pallas import tpu_sc as plsc

D_MODEL = 1024
NC = 2
NS = 16
L = 16
NW = NC * NS
SCALE = 32.0


@functools.lru_cache(maxsize=None)
def _make_kernel(BT: int, S: int, D: int, K: int):
    b_per_w = BT // NW
    nch = b_per_w // K
    mesh = plsc.VectorSubcoreMesh(
        core_axis_name="c", subcore_axis_name="s", num_cores=NC, num_subcores=NS
    )

    @functools.partial(
        pl.kernel,
        out_type=jax.ShapeDtypeStruct((BT, D), jnp.float32),
        mesh=mesh,
        scratch_types=[
            pltpu.VMEM((nch, K), jnp.int32),
            pltpu.VMEM((K, D), jnp.float32),
            pltpu.VMEM((K, D), jnp.float32),
            pltpu.SemaphoreType.DMA,
        ],
    )
    def emb_kernel(idx_hbm, table_hbm, pos_hbm, out_hbm, idx_v, gbuf, pbuf, sem):
        wid = lax.axis_index("s") * NC + lax.axis_index("c")
        base = wid * b_per_w
        s0 = lax.rem(base, S)
        pltpu.sync_copy(idx_hbm.at[wid], idx_v)

        def chunk(c, carry):
            pltpu.async_copy(table_hbm.at[idx_v.at[c]], gbuf, sem).wait()
            pltpu.sync_copy(pos_hbm.at[pl.ds(s0 + c * K, K)], pbuf)

            def row(i, rcarry):
                for j in range(D // L):
                    sl = pl.ds(j * L, L)
                    gbuf[i, sl] = gbuf[i, sl] * SCALE + pbuf[i, sl]
                return rcarry

            lax.fori_loop(0, K, row, 0)
            pltpu.sync_copy(gbuf, out_hbm.at[pl.ds(base + c * K, K)])
            return carry

        lax.fori_loop(0, nch, chunk, 0)

    return emb_kernel


def kernel(input_token_vec, table, pos_encoding):
    B, S = input_token_vec.shape
    BT = B * S
    K = 16
    idx = input_token_vec.reshape(NW, BT // (NW * K), K)
    out = _make_kernel(BT, S, D_MODEL, K)(idx, table, pos_encoding)
    return out.reshape(B, S, D_MODEL)


# speedup vs baseline: 1.3552x; 1.3552x over previous
import functools

import jax
import jax.numpy as jnp
from jax import lax
from jax.experimental import pallas as pl
from jax.experimental.pallas import tpu as pltpu
from jax.experimental.pallas import tpu_sc as plsc

D_MODEL = 1024
NC = 2
NS = 16
L = 16
NW = NC * NS
SCALE = 32.0
K = 16
NBUF = 3
G = 2


@functools.lru_cache(maxsize=None)
def _make_kernel(B: int, S: int, D: int):
    W = S // NW
    CB = W // K
    nch = B * CB
    mesh = plsc.VectorSubcoreMesh(
        core_axis_name="c", subcore_axis_name="s", num_cores=NC, num_subcores=NS
    )

    @functools.partial(
        pl.kernel,
        out_type=jax.ShapeDtypeStruct((B * S, D), jnp.float32),
        mesh=mesh,
        scratch_types=[
            pltpu.VMEM((nch, K), jnp.int32),
            pltpu.VMEM((W, D), jnp.float32),
            pltpu.VMEM((K, D), jnp.float32),
            pltpu.VMEM((K, D), jnp.float32),
            pltpu.VMEM((K, D), jnp.float32),
            pltpu.SemaphoreType.DMA,
            pltpu.SemaphoreType.DMA,
            pltpu.SemaphoreType.DMA,
            pltpu.SemaphoreType.DMA,
            pltpu.SemaphoreType.DMA,
            pltpu.SemaphoreType.DMA,
            pltpu.SemaphoreType.DMA,
        ],
    )
    def emb_kernel(idx_hbm, table_hbm, pos_hbm, out_hbm,
                   idx_v, pbuf, g0, g1, g2,
                   psem, gs0, gs1, gs2, ss0, ss1, ss2):
        gb = (g0, g1, g2)
        gsem = (gs0, gs1, gs2)
        ssem = (ss0, ss1, ss2)
        wid = lax.axis_index("s") * NC + lax.axis_index("c")
        pltpu.sync_copy(idx_hbm.at[wid], idx_v)
        pcopy = pltpu.async_copy(pos_hbm.at[pl.ds(wid * W, W)], pbuf, psem)

        hg = [None] * nch
        hs = [None] * nch
        for n in range(G):
            hg[n] = pltpu.async_copy(
                table_hbm.at[idx_v.at[n]], gb[n % NBUF], gsem[n % NBUF])

        for c in range(nch):
            p = c % NBUF
            hg[c].wait()
            if c == 0:
                pcopy.wait()
            poff = (c % CB) * K

            def row(i, carry, _p=p, _poff=poff):
                for t in range(D // L):
                    sl = pl.ds(t * L, L)
                    gb[_p][i, sl] = gb[_p][i, sl] * SCALE + pbuf[_poff + i, sl]
                return carry

            lax.fori_loop(0, K, row, 0)
            flat = (c // CB) * S + wid * W + poff
            hs[c] = pltpu.async_copy(gb[p], out_hbm.at[pl.ds(flat, K)], ssem[p])

            n = c + G
            if n < nch:
                if n >= NBUF:
                    hs[n - NBUF].wait()
                hg[n] = pltpu.async_copy(
                    table_hbm.at[idx_v.at[n]], gb[n % NBUF], gsem[n % NBUF])

        for c in range(nch - NBUF, nch):
            hs[c].wait()

    return emb_kernel


def kernel(input_token_vec, table, pos_encoding):
    B, S = input_token_vec.shape
    W = S // NW
    CB = W // K
    idx = (input_token_vec.reshape(B, NW, CB, K)
           .transpose(1, 0, 2, 3)
           .reshape(NW, B * CB, K))
    out = _make_kernel(B, S, D_MODEL)(idx, table, pos_encoding)
    return out.reshape(B, S, D_MODEL)


# speedup vs baseline: 1.5552x; 1.1476x over previous
import functools

import jax
import jax.numpy as jnp
from jax import lax
from jax.experimental import pallas as pl
from jax.experimental.pallas import tpu as pltpu
from jax.experimental.pallas import tpu_sc as plsc

D_MODEL = 1024
NC = 2
NS = 16
L = 16
NW = NC * NS
SCALE = 32.0
P = 8
NBUF = 3
G = 2
Q = 16


@functools.lru_cache(maxsize=None)
def _make_kernel(B: int, S: int, D: int):
    W = S // NW
    nwin = W // P
    rows = B * P
    nq = D // (Q * L)
    mesh = plsc.VectorSubcoreMesh(
        core_axis_name="c", subcore_axis_name="s", num_cores=NC, num_subcores=NS
    )

    @functools.partial(
        pl.kernel,
        out_type=jax.ShapeDtypeStruct((B * S, D), jnp.float32),
        mesh=mesh,
        scratch_types=[
            pltpu.VMEM((nwin, rows), jnp.int32),
            pltpu.VMEM((rows, D), jnp.float32),
            pltpu.VMEM((rows, D), jnp.float32),
            pltpu.VMEM((rows, D), jnp.float32),
            pltpu.VMEM((P, D), jnp.float32),
            pltpu.VMEM((P, D), jnp.float32),
            pltpu.VMEM((P, D), jnp.float32),
            pltpu.SemaphoreType.DMA,
            pltpu.SemaphoreType.DMA,
            pltpu.SemaphoreType.DMA,
            pltpu.SemaphoreType.DMA,
            pltpu.SemaphoreType.DMA,
            pltpu.SemaphoreType.DMA,
            pltpu.SemaphoreType.DMA,
            pltpu.SemaphoreType.DMA,
            pltpu.SemaphoreType.DMA,
        ],
    )
    def emb_kernel(idx_hbm, table_hbm, pos_hbm, out_hbm,
                   idx_v, g0, g1, g2, p0, p1, p2,
                   gs0, gs1, gs2, ps0, ps1, ps2, ss0, ss1, ss2):
        gb = (g0, g1, g2)
        pb = (p0, p1, p2)
        gsem = (gs0, gs1, gs2)
        psem = (ps0, ps1, ps2)
        ssem = (ss0, ss1, ss2)
        wid = lax.axis_index("s") * NC + lax.axis_index("c")
        pltpu.sync_copy(idx_hbm.at[wid], idx_v)

        def start_window(n):
            q = n % NBUF
            hg = pltpu.async_copy(table_hbm.at[idx_v.at[n]], gb[q], gsem[q])
            hp = pltpu.async_copy(
                pos_hbm.at[pl.ds(wid * W + n * P, P)], pb[q], psem[q])
            return hg, hp

        hg = [None] * nwin
        hp = [None] * nwin
        hs = [None] * nwin
        for n in range(G):
            hg[n], hp[n] = start_window(n)

        for j in range(nwin):
            q = j % NBUF
            hg[j].wait()
            hp[j].wait()

            def row(i, carry, _q=q):
                def quarter(h, carry2):
                    base = h * (Q * L)
                    pv = [pb[_q][i, pl.ds(base + t * L, L)] for t in range(Q)]
                    for b in range(B):
                        r = b * P + i
                        for t in range(Q):
                            sl = pl.ds(base + t * L, L)
                            gb[_q][r, sl] = gb[_q][r, sl] * SCALE + pv[t]
                    return carry2

                return lax.fori_loop(0, nq, quarter, carry)

            lax.fori_loop(0, P, row, 0)

            hs[j] = [
                pltpu.async_copy(
                    gb[q].at[pl.ds(b * P, P)],
                    out_hbm.at[pl.ds(b * S + wid * W + j * P, P)],
                    ssem[q],
                )
                for b in range(B)
            ]

            n = j + G
            if n < nwin:
                if n >= NBUF:
                    for h in hs[n - NBUF]:
                        h.wait()
                hg[n], hp[n] = start_window(n)

        for c in range(nwin - NBUF, nwin):
            for h in hs[c]:
                h.wait()

    return emb_kernel


def kernel(input_token_vec, table, pos_encoding):
    B, S = input_token_vec.shape
    W = S // NW
    nwin = W // P
    idx = (input_token_vec.reshape(B, NW, nwin, P)
           .transpose(1, 2, 0, 3)
           .reshape(NW, nwin, B * P))
    out = _make_kernel(B, S, D_MODEL)(idx, table, pos_encoding)
    return out.reshape(B, S, D_MODEL)
